# baseline (speedup 1.0000x reference)
import numpy as np
import jax
import jax.numpy as jnp
from jax.experimental import pallas as pl
from jax.experimental.pallas import tpu as pltpu

_LEVELS = np.array([8, 5, 5, 5, 5], dtype=np.int64)
_BASIS = np.cumprod(np.concatenate([[1], _LEVELS[:-1]])).astype(np.int64)
_K = int(np.prod(_LEVELS))
_KP = 5120
_D = len(_LEVELS)
_DP = 8
_INV_TEMP = 100.0
_EPS_BOUND = 1e-3
_LOG_EPS = float(np.log(1e-5))
_ENT_W = 0.1
_GAMMA = 1.0
_NEG = -1e30


def _pad_row(x, fill=0.0):
    out = np.full((1, _DP), fill, np.float32)
    out[0, : _D] = x
    return out


_lv = _LEVELS.astype(np.float64)
_half_l_np = (_lv - 1.0) * (1.0 + _EPS_BOUND) / 2.0
_offset_np = np.where(_LEVELS % 2 == 0, 0.5, 0.0)
_shift_np = np.arctanh(_offset_np / _half_l_np)
_hw_np = (_LEVELS // 2).astype(np.float64)

_SHIFT = _pad_row(_shift_np)
_HALF_L = _pad_row(_half_l_np, 1.0)
_OFFSET = _pad_row(_offset_np)
_HW = _pad_row(_hw_np)
_INV_HW = _pad_row(1.0 / _hw_np, 1.0)
_BASIS_F = _pad_row(_BASIS)

_CS = np.zeros((_DP, 8), np.float32)
_MS = np.full((_DP, 8), _NEG, np.float32)
for _d in range(_D):
    _L = int(_LEVELS[_d])
    _h = _L // 2
    for _j in range(_L):
        _CS[_d, _j] = (_j - _h) / _h
        _MS[_d, _j] = 0.0
for _d in range(_D, _DP):
    _MS[_d, 0] = 0.0

_CBT = np.zeros((_DP, _KP), np.float32)
_kk = np.arange(_K)
for _d in range(_D):
    _L = int(_LEVELS[_d])
    _h = _L // 2
    _CBT[_d, : _K] = (2.0 * _INV_TEMP) * ((_kk // int(_BASIS[_d])) % _L - _h) / _h
_CBT[_D, _K:] = _NEG

_T = 256
_NTOK = 4096


_QC = np.zeros((8, _DP), np.float32)
_QC[0] = _SHIFT
_QC[1] = _HALF_L
_QC[2] = _OFFSET
_QC[3] = _HW
_QC[4] = _INV_HW
_QC[5] = _BASIS_F


def _body(z_ref, winT_ref, bin_ref, cbT_ref, woutT_ref, bout_ref,
          qc_ref, cs_ref, ms_ref,
          out_ref, idx_ref, aux_ref, psum_ref, ent_ref):
    i = pl.program_id(0)

    @pl.when(i == 0)
    def _init():
        psum_ref[...] = jnp.zeros_like(psum_ref)
        ent_ref[0] = 0.0

    z = z_ref[...]
    zp = jnp.dot(z, winT_ref[...], preferred_element_type=jnp.float32)
    zp = zp + bin_ref[...]

    qc = qc_ref[...]
    shift, half_l, offset = qc[0:1], qc[1:2], qc[2:3]
    hw, inv_hw, basis_f = qc[3:4], qc[4:5], qc[5:6]
    bounded = jnp.tanh(zp + shift) * half_l - offset
    r = jnp.round(bounded)
    codes = r * inv_hw
    idxf = jnp.sum((r + hw) * basis_f, axis=1, keepdims=True)
    idx_ref[...] = idxf.astype(jnp.int32)
    out = jnp.dot(codes.astype(jnp.bfloat16), woutT_ref[...],
                  preferred_element_type=jnp.float32)
    out_ref[...] = out + bout_ref[...]

    zpb = zp.astype(jnp.bfloat16)
    zpf = zpb.astype(jnp.float32)
    cs = cs_ref[...]
    ms = ms_ref[...]
    small = (2.0 * _INV_TEMP) * zpf[:, :, None] * cs[None] + ms[None]
    md = jnp.max(small, axis=2)
    m = jnp.sum(md, axis=1, keepdims=True)
    zd = jnp.sum(jnp.exp(small - md[:, :, None]), axis=2)
    t = jnp.sum(jnp.log(zd), axis=1, keepdims=True)
    mt = m + t

    logits = jnp.dot(zpb, cbT_ref[...], preferred_element_type=jnp.float32)
    u = logits - mt
    p = jnp.exp(u)
    v = p * jnp.maximum(u, _LOG_EPS)
    ent_ref[0] += -jnp.sum(v)
    psum_ref[...] += jnp.sum(p, axis=0, keepdims=True)

    @pl.when(i == pl.num_programs(0) - 1)
    def _fini():
        pse = ent_ref[0] / _NTOK
        ap = psum_ref[...] * (1.0 / _NTOK)
        ce = jnp.sum(-ap * jnp.log(jnp.maximum(ap, 1e-5)))
        val = _ENT_W * (pse - _GAMMA * ce)
        aux_ref[...] = jnp.broadcast_to(val, (1, 1))


def kernel(z, W_in, b_in, W_out, b_out):
    b, n, dim = z.shape
    ntok = b * n
    zf = z.reshape(ntok, dim).astype(jnp.bfloat16)
    winT = jnp.zeros((dim, _DP), jnp.bfloat16).at[:, : _D].set(
        W_in.T.astype(jnp.bfloat16))
    binp = jnp.zeros((1, _DP), jnp.float32).at[0, : _D].set(b_in)
    binp = binp.at[0, _D].set(1.0)
    woutT = jnp.zeros((_DP, dim), jnp.bfloat16).at[: _D, :].set(
        W_out.T.astype(jnp.bfloat16))
    boutp = b_out.reshape(1, dim)
    cbT = jnp.asarray(_CBT, jnp.bfloat16)

    grid = ntok // _T
    out, idx, aux = pl.pallas_call(
        _body,
        grid=(grid,),
        in_specs=[
            pl.BlockSpec((_T, dim), lambda i: (i, 0)),
            pl.BlockSpec((dim, _DP), lambda i: (0, 0)),
            pl.BlockSpec((1, _DP), lambda i: (0, 0)),
            pl.BlockSpec((_DP, _KP), lambda i: (0, 0)),
            pl.BlockSpec((_DP, dim), lambda i: (0, 0)),
            pl.BlockSpec((1, dim), lambda i: (0, 0)),
            pl.BlockSpec((8, _DP), lambda i: (0, 0)),
            pl.BlockSpec((_DP, 8), lambda i: (0, 0)),
            pl.BlockSpec((_DP, 8), lambda i: (0, 0)),
        ],
        out_specs=[
            pl.BlockSpec((_T, dim), lambda i: (i, 0)),
            pl.BlockSpec((_T, 1), lambda i: (i, 0)),
            pl.BlockSpec((1, 1), lambda i: (0, 0)),
        ],
        out_shape=[
            jax.ShapeDtypeStruct((ntok, dim), jnp.float32),
            jax.ShapeDtypeStruct((ntok, 1), jnp.int32),
            jax.ShapeDtypeStruct((1, 1), jnp.float32),
        ],
        scratch_shapes=[
            pltpu.VMEM((1, _KP), jnp.float32),
            pltpu.SMEM((1,), jnp.float32),
        ],
        compiler_params=pltpu.CompilerParams(
            dimension_semantics=("arbitrary",)),
    )(zf, winT, binp, cbT, woutT, boutp,
      jnp.asarray(_QC), jnp.asarray(_CS), jnp.asarray(_MS))
    return out.reshape(b, n, dim), idx.reshape(b, n), aux[0, 0]


# speedup vs baseline: 1.2653x; 1.2653x over previous
import numpy as np
import jax
import jax.numpy as jnp
from jax.experimental import pallas as pl
from jax.experimental.pallas import tpu as pltpu

_LEVELS = np.array([8, 5, 5, 5, 5], dtype=np.int64)
_BASIS = np.cumprod(np.concatenate([[1], _LEVELS[:-1]])).astype(np.int64)
_K = int(np.prod(_LEVELS))
_KP = 5120
_D = len(_LEVELS)
_DP = 8
_INV_TEMP = 100.0
_EPS_BOUND = 1e-3
_LOG_EPS = float(np.log(1e-5))
_ENT_W = 0.1
_GAMMA = 1.0
_NEG = -1e30


def _pad_row(x, fill=0.0):
    out = np.full((1, _DP), fill, np.float32)
    out[0, : _D] = x
    return out


_lv = _LEVELS.astype(np.float64)
_half_l_np = (_lv - 1.0) * (1.0 + _EPS_BOUND) / 2.0
_offset_np = np.where(_LEVELS % 2 == 0, 0.5, 0.0)
_shift_np = np.arctanh(_offset_np / _half_l_np)
_hw_np = (_LEVELS // 2).astype(np.float64)

_SHIFT = _pad_row(_shift_np)
_HALF_L = _pad_row(_half_l_np, 1.0)
_OFFSET = _pad_row(_offset_np)
_HW = _pad_row(_hw_np)
_INV_HW = _pad_row(1.0 / _hw_np, 1.0)
_BASIS_F = _pad_row(_BASIS)

_CS = np.zeros((_DP, 8), np.float32)
_MS = np.full((_DP, 8), _NEG, np.float32)
for _d in range(_D):
    _L = int(_LEVELS[_d])
    _h = _L // 2
    for _j in range(_L):
        _CS[_d, _j] = (_j - _h) / _h
        _MS[_d, _j] = 0.0
for _d in range(_D, _DP):
    _MS[_d, 0] = 0.0

_CBT = np.zeros((_DP, _KP), np.float32)
_kk = np.arange(_K)
for _d in range(_D):
    _L = int(_LEVELS[_d])
    _h = _L // 2
    _CBT[_d, : _K] = (2.0 * _INV_TEMP) * ((_kk // int(_BASIS[_d])) % _L - _h) / _h
_CBT[_D, _K:] = _NEG

_T = 256
_NTOK = 4096


_QC = np.zeros((8, _DP), np.float32)
_QC[0] = _SHIFT
_QC[1] = _HALF_L
_QC[2] = _OFFSET
_QC[3] = _HW
_QC[4] = _INV_HW
_QC[5] = _BASIS_F


def _body(z_ref, winT_ref, bin_ref, cbT_ref, woutT_ref, bout_ref,
          qc_ref, cs_ref, ms_ref,
          out_ref, idx_ref, aux_ref, psum_ref, ent_ref):
    i = pl.program_id(0)

    @pl.when(i == 0)
    def _init():
        psum_ref[...] = jnp.zeros_like(psum_ref)
        ent_ref[0] = 0.0

    z = z_ref[...]
    zp = jnp.dot(z, winT_ref[...], preferred_element_type=jnp.float32)
    zp = zp + bin_ref[...]

    qc = qc_ref[...]
    shift, half_l, offset = qc[0:1], qc[1:2], qc[2:3]
    hw, inv_hw, basis_f = qc[3:4], qc[4:5], qc[5:6]
    bounded = jnp.tanh(zp + shift) * half_l - offset
    r = jnp.round(bounded)
    codes = r * inv_hw
    idxf = jnp.sum((r + hw) * basis_f, axis=1, keepdims=True)
    idx_ref[...] = idxf.astype(jnp.int32)
    out = jnp.dot(codes.astype(jnp.bfloat16), woutT_ref[...],
                  preferred_element_type=jnp.float32)
    out_ref[...] = out + bout_ref[...]

    zpb = zp.astype(jnp.bfloat16)
    zpf = zpb.astype(jnp.float32)
    cs = cs_ref[...]
    ms = ms_ref[...]
    small = (2.0 * _INV_TEMP) * zpf[:, :, None] * cs[None] + ms[None]
    md = jnp.max(small, axis=2)
    m = jnp.sum(md, axis=1, keepdims=True)
    zd = jnp.sum(jnp.exp(small - md[:, :, None]), axis=2)
    t = jnp.sum(jnp.log(zd), axis=1, keepdims=True)
    mt = m + t

    logits = jnp.dot(zpb, cbT_ref[...], preferred_element_type=jnp.float32)
    u = logits - mt
    p = jnp.exp(u)
    v = p * jnp.maximum(u, _LOG_EPS)
    rowv = jnp.sum(v, axis=1, keepdims=True)
    ent_ref[0] += -jnp.sum(rowv)
    ones = jnp.ones((_T, 1), jnp.float32)
    pblk = jax.lax.dot_general(ones, p, (((0,), (0,)), ((), ())),
                               preferred_element_type=jnp.float32)
    psum_ref[...] += pblk

    @pl.when(i == pl.num_programs(0) - 1)
    def _fini():
        pse = ent_ref[0] / _NTOK
        ap = psum_ref[...] * (1.0 / _NTOK)
        ce = jnp.sum(-ap * jnp.log(jnp.maximum(ap, 1e-5)))
        val = _ENT_W * (pse - _GAMMA * ce)
        aux_ref[...] = jnp.broadcast_to(val, (1, 1))


def kernel(z, W_in, b_in, W_out, b_out):
    b, n, dim = z.shape
    ntok = b * n
    zf = z.reshape(ntok, dim).astype(jnp.bfloat16)
    winT = jnp.zeros((dim, _DP), jnp.bfloat16).at[:, : _D].set(
        W_in.T.astype(jnp.bfloat16))
    binp = jnp.zeros((1, _DP), jnp.float32).at[0, : _D].set(b_in)
    binp = binp.at[0, _D].set(1.0)
    woutT = jnp.zeros((_DP, dim), jnp.bfloat16).at[: _D, :].set(
        W_out.T.astype(jnp.bfloat16))
    boutp = b_out.reshape(1, dim)
    cbT = jnp.asarray(_CBT, jnp.bfloat16)

    grid = ntok // _T
    out, idx, aux = pl.pallas_call(
        _body,
        grid=(grid,),
        in_specs=[
            pl.BlockSpec((_T, dim), lambda i: (i, 0)),
            pl.BlockSpec((dim, _DP), lambda i: (0, 0)),
            pl.BlockSpec((1, _DP), lambda i: (0, 0)),
            pl.BlockSpec((_DP, _KP), lambda i: (0, 0)),
            pl.BlockSpec((_DP, dim), lambda i: (0, 0)),
            pl.BlockSpec((1, dim), lambda i: (0, 0)),
            pl.BlockSpec((8, _DP), lambda i: (0, 0)),
            pl.BlockSpec((_DP, 8), lambda i: (0, 0)),
            pl.BlockSpec((_DP, 8), lambda i: (0, 0)),
        ],
        out_specs=[
            pl.BlockSpec((_T, dim), lambda i: (i, 0)),
            pl.BlockSpec((_T, 1), lambda i: (i, 0)),
            pl.BlockSpec((1, 1), lambda i: (0, 0)),
        ],
        out_shape=[
            jax.ShapeDtypeStruct((ntok, dim), jnp.float32),
            jax.ShapeDtypeStruct((ntok, 1), jnp.int32),
            jax.ShapeDtypeStruct((1, 1), jnp.float32),
        ],
        scratch_shapes=[
            pltpu.VMEM((1, _KP), jnp.float32),
            pltpu.SMEM((1,), jnp.float32),
        ],
        compiler_params=pltpu.CompilerParams(
            dimension_semantics=("arbitrary",)),
    )(zf, winT, binp, cbT, woutT, boutp,
      jnp.asarray(_QC), jnp.asarray(_CS), jnp.asarray(_MS))
    return out.reshape(b, n, dim), idx.reshape(b, n), aux[0, 0]


# speedup vs baseline: 1.4034x; 1.1091x over previous
import numpy as np
import jax
import jax.numpy as jnp
from jax.experimental import pallas as pl
from jax.experimental.pallas import tpu as pltpu

_LEVELS = np.array([8, 5, 5, 5, 5], dtype=np.int64)
_BASIS = np.cumprod(np.concatenate([[1], _LEVELS[:-1]])).astype(np.int64)
_K = int(np.prod(_LEVELS))
_KP = 5120
_D = len(_LEVELS)
_DP = 8
_INV_TEMP = 100.0
_EPS_BOUND = 1e-3
_LOG_EPS = float(np.log(1e-5))
_ENT_W = 0.1
_GAMMA = 1.0
_NEG = -1e30


def _pad_row(x, fill=0.0):
    out = np.full((1, _DP), fill, np.float32)
    out[0, : _D] = x
    return out


_lv = _LEVELS.astype(np.float64)
_half_l_np = (_lv - 1.0) * (1.0 + _EPS_BOUND) / 2.0
_offset_np = np.where(_LEVELS % 2 == 0, 0.5, 0.0)
_shift_np = np.arctanh(_offset_np / _half_l_np)
_hw_np = (_LEVELS // 2).astype(np.float64)

_SHIFT = _pad_row(_shift_np)
_HALF_L = _pad_row(_half_l_np, 1.0)
_OFFSET = _pad_row(_offset_np)
_HW = _pad_row(_hw_np)
_INV_HW = _pad_row(1.0 / _hw_np, 1.0)
_BASIS_F = _pad_row(_BASIS)

_CS = np.zeros((_DP, 8), np.float32)
_MS = np.full((_DP, 8), _NEG, np.float32)
for _d in range(_D):
    _L = int(_LEVELS[_d])
    _h = _L // 2
    for _j in range(_L):
        _CS[_d, _j] = (_j - _h) / _h
        _MS[_d, _j] = 0.0
for _d in range(_D, _DP):
    _MS[_d, 0] = 0.0

_CBT = np.zeros((_DP, _KP), np.float32)
_kk = np.arange(_K)
for _d in range(_D):
    _L = int(_LEVELS[_d])
    _h = _L // 2
    _CBT[_d, : _K] = (2.0 * _INV_TEMP) * ((_kk // int(_BASIS[_d])) % _L - _h) / _h
_CBT[_D, _K:] = _NEG

_T = 256
_NTOK = 4096


_QC = np.zeros((8, _DP), np.float32)
_QC[0] = _SHIFT
_QC[1] = _HALF_L
_QC[2] = _OFFSET
_QC[3] = _HW
_QC[4] = _INV_HW
_QC[5] = _BASIS_F


def _body(z_ref, winT_ref, bin_ref, cbT_ref, woutT_ref, bout_ref,
          qc_ref, cs_ref, ms_ref,
          out_ref, idx_ref, aux_ref, psum_ref, ent_ref):
    i = pl.program_id(0)

    @pl.when(i == 0)
    def _init():
        psum_ref[...] = jnp.zeros_like(psum_ref)
        ent_ref[0] = 0.0

    z = z_ref[...].astype(jnp.bfloat16)
    zp = jnp.dot(z, winT_ref[...], preferred_element_type=jnp.float32)
    zp = zp + bin_ref[...]

    qc = qc_ref[...]
    shift, half_l, offset = qc[0:1], qc[1:2], qc[2:3]
    hw, inv_hw, basis_f = qc[3:4], qc[4:5], qc[5:6]
    bounded = jnp.tanh(zp + shift) * half_l - offset
    r = jnp.round(bounded)
    codes = r * inv_hw
    idxf = jnp.sum((r + hw) * basis_f, axis=1, keepdims=True)
    idx_ref[...] = idxf.astype(jnp.int32)
    out = jnp.dot(codes.astype(jnp.bfloat16), woutT_ref[...],
                  preferred_element_type=jnp.float32)
    out_ref[...] = out + bout_ref[...]

    zpb = zp.astype(jnp.bfloat16)
    zpf = zpb.astype(jnp.float32)
    cs = cs_ref[...]
    ms = ms_ref[...]
    small = (2.0 * _INV_TEMP) * zpf[:, :, None] * cs[None] + ms[None]
    md = jnp.max(small, axis=2)
    m = jnp.sum(md, axis=1, keepdims=True)
    zd = jnp.sum(jnp.exp(small - md[:, :, None]), axis=2)
    t = jnp.sum(jnp.log(zd), axis=1, keepdims=True)
    mt = m + t

    logits = jnp.dot(zpb, cbT_ref[...], preferred_element_type=jnp.float32)
    u = logits - mt
    p = jnp.exp(u)
    v = p * jnp.maximum(u, _LOG_EPS)
    rowv = jnp.sum(v, axis=1, keepdims=True)
    ent_ref[0] += -jnp.sum(rowv)
    ones = jnp.ones((_T, 1), jnp.float32)
    pblk = jax.lax.dot_general(ones, p, (((0,), (0,)), ((), ())),
                               preferred_element_type=jnp.float32)
    psum_ref[...] += pblk

    @pl.when(i == pl.num_programs(0) - 1)
    def _fini():
        pse = ent_ref[0] / _NTOK
        ap = psum_ref[...] * (1.0 / _NTOK)
        ce = jnp.sum(-ap * jnp.log(jnp.maximum(ap, 1e-5)))
        val = _ENT_W * (pse - _GAMMA * ce)
        aux_ref[...] = jnp.broadcast_to(val, (1, 1))


def kernel(z, W_in, b_in, W_out, b_out):
    b, n, dim = z.shape
    ntok = b * n
    zf = z.reshape(ntok, dim)
    winT = jnp.zeros((dim, _DP), jnp.bfloat16).at[:, : _D].set(
        W_in.T.astype(jnp.bfloat16))
    binp = jnp.zeros((1, _DP), jnp.float32).at[0, : _D].set(b_in)
    binp = binp.at[0, _D].set(1.0)
    woutT = jnp.zeros((_DP, dim), jnp.bfloat16).at[: _D, :].set(
        W_out.T.astype(jnp.bfloat16))
    boutp = b_out.reshape(1, dim)
    cbT = jnp.asarray(_CBT, jnp.bfloat16)

    grid = ntok // _T
    out, idx, aux = pl.pallas_call(
        _body,
        grid=(grid,),
        in_specs=[
            pl.BlockSpec((_T, dim), lambda i: (i, 0)),
            pl.BlockSpec((dim, _DP), lambda i: (0, 0)),
            pl.BlockSpec((1, _DP), lambda i: (0, 0)),
            pl.BlockSpec((_DP, _KP), lambda i: (0, 0)),
            pl.BlockSpec((_DP, dim), lambda i: (0, 0)),
            pl.BlockSpec((1, dim), lambda i: (0, 0)),
            pl.BlockSpec((8, _DP), lambda i: (0, 0)),
            pl.BlockSpec((_DP, 8), lambda i: (0, 0)),
            pl.BlockSpec((_DP, 8), lambda i: (0, 0)),
        ],
        out_specs=[
            pl.BlockSpec((_T, dim), lambda i: (i, 0)),
            pl.BlockSpec((_T, 1), lambda i: (i, 0)),
            pl.BlockSpec((1, 1), lambda i: (0, 0)),
        ],
        out_shape=[
            jax.ShapeDtypeStruct((ntok, dim), jnp.float32),
            jax.ShapeDtypeStruct((ntok, 1), jnp.int32),
            jax.ShapeDtypeStruct((1, 1), jnp.float32),
        ],
        scratch_shapes=[
            pltpu.VMEM((1, _KP), jnp.float32),
            pltpu.SMEM((1,), jnp.float32),
        ],
        compiler_params=pltpu.CompilerParams(
            dimension_semantics=("arbitrary",)),
    )(zf, winT, binp, cbT, woutT, boutp,
      jnp.asarray(_QC), jnp.asarray(_CS), jnp.asarray(_MS))
    return out.reshape(b, n, dim), idx.reshape(b, n), aux[0, 0]


# speedup vs baseline: 1.4438x; 1.0288x over previous
import numpy as np
import jax
import jax.numpy as jnp
from jax.experimental import pallas as pl
from jax.experimental.pallas import tpu as pltpu

_LEVELS = np.array([8, 5, 5, 5, 5], dtype=np.int64)
_BASIS = np.cumprod(np.concatenate([[1], _LEVELS[:-1]])).astype(np.int64)
_K = int(np.prod(_LEVELS))
_KP = 5120
_D = len(_LEVELS)
_DP = 8
_INV_TEMP = 100.0
_EPS_BOUND = 1e-3
_LOG_EPS = float(np.log(1e-5))
_ENT_W = 0.1
_GAMMA = 1.0
_NEG = -1e30


def _pad_row(x, fill=0.0):
    out = np.full((1, _DP), fill, np.float32)
    out[0, : _D] = x
    return out


_lv = _LEVELS.astype(np.float64)
_half_l_np = (_lv - 1.0) * (1.0 + _EPS_BOUND) / 2.0
_offset_np = np.where(_LEVELS % 2 == 0, 0.5, 0.0)
_shift_np = np.arctanh(_offset_np / _half_l_np)
_hw_np = (_LEVELS // 2).astype(np.float64)

_SHIFT = _pad_row(_shift_np)
_HALF_L = _pad_row(_half_l_np, 1.0)
_OFFSET = _pad_row(_offset_np)
_HW = _pad_row(_hw_np)
_INV_HW = _pad_row(1.0 / _hw_np, 1.0)
_BASIS_F = _pad_row(_BASIS)

_CS = np.zeros((_DP, 8), np.float32)
_MS = np.full((_DP, 8), _NEG, np.float32)
for _d in range(_D):
    _L = int(_LEVELS[_d])
    _h = _L // 2
    for _j in range(_L):
        _CS[_d, _j] = (_j - _h) / _h
        _MS[_d, _j] = 0.0
for _d in range(_D, _DP):
    _MS[_d, 0] = 0.0

_CBT = np.zeros((_DP, _KP), np.float32)
_kk = np.arange(_K)
for _d in range(_D):
    _L = int(_LEVELS[_d])
    _h = _L // 2
    _CBT[_d, : _K] = (2.0 * _INV_TEMP) * ((_kk // int(_BASIS[_d])) % _L - _h) / _h
_CBT[_D, _K:] = _NEG

_T = 512
_NTOK = 4096


_QC = np.zeros((8, _DP), np.float32)
_QC[0] = _SHIFT
_QC[1] = _HALF_L
_QC[2] = _OFFSET
_QC[3] = _HW
_QC[4] = _INV_HW
_QC[5] = _BASIS_F


def _body(z_ref, winT_ref, bin_ref, cbT_ref, woutT_ref, bout_ref,
          qc_ref, cs_ref, ms_ref,
          out_ref, idx_ref, aux_ref, psum_ref, ent_ref):
    i = pl.program_id(0)

    @pl.when(i == 0)
    def _init():
        psum_ref[...] = jnp.zeros_like(psum_ref)
        ent_ref[0] = 0.0

    z = z_ref[...].astype(jnp.bfloat16)
    zp = jnp.dot(z, winT_ref[...], preferred_element_type=jnp.float32)
    zp = zp + bin_ref[...]

    qc = qc_ref[...]
    shift, half_l, offset = qc[0:1], qc[1:2], qc[2:3]
    hw, inv_hw, basis_f = qc[3:4], qc[4:5], qc[5:6]
    bounded = jnp.tanh(zp + shift) * half_l - offset
    r = jnp.round(bounded)
    codes = r * inv_hw
    idxf = jnp.sum((r + hw) * basis_f, axis=1, keepdims=True)
    idx_ref[...] = idxf.astype(jnp.int32)
    out = jnp.dot(codes.astype(jnp.bfloat16), woutT_ref[...],
                  preferred_element_type=jnp.float32)
    out_ref[...] = out + bout_ref[...]

    zpb = zp.astype(jnp.bfloat16)
    zpf = zpb.astype(jnp.float32)
    cs = cs_ref[...]
    ms = ms_ref[...]
    small = (2.0 * _INV_TEMP) * zpf[:, :, None] * cs[None] + ms[None]
    md = jnp.max(small, axis=2)
    m = jnp.sum(md, axis=1, keepdims=True)
    zd = jnp.sum(jnp.exp(small - md[:, :, None]), axis=2)
    t = jnp.sum(jnp.log(zd), axis=1, keepdims=True)
    mt = m + t

    logits = jnp.dot(zpb, cbT_ref[...], preferred_element_type=jnp.float32)
    u = logits - mt
    p = jnp.exp(u)
    v = p * jnp.maximum(u, _LOG_EPS)
    rowv = jnp.sum(v, axis=1, keepdims=True)
    ent_ref[0] += -jnp.sum(rowv)
    ones = jnp.ones((_T, 1), jnp.float32)
    pblk = jax.lax.dot_general(ones, p, (((0,), (0,)), ((), ())),
                               preferred_element_type=jnp.float32)
    psum_ref[...] += pblk

    @pl.when(i == pl.num_programs(0) - 1)
    def _fini():
        pse = ent_ref[0] / _NTOK
        ap = psum_ref[...] * (1.0 / _NTOK)
        ce = jnp.sum(-ap * jnp.log(jnp.maximum(ap, 1e-5)))
        val = _ENT_W * (pse - _GAMMA * ce)
        aux_ref[...] = jnp.broadcast_to(val, (1, 1))


def kernel(z, W_in, b_in, W_out, b_out):
    b, n, dim = z.shape
    ntok = b * n
    zf = z.reshape(ntok, dim)
    winT = jnp.zeros((dim, _DP), jnp.bfloat16).at[:, : _D].set(
        W_in.T.astype(jnp.bfloat16))
    binp = jnp.zeros((1, _DP), jnp.float32).at[0, : _D].set(b_in)
    binp = binp.at[0, _D].set(1.0)
    woutT = jnp.zeros((_DP, dim), jnp.bfloat16).at[: _D, :].set(
        W_out.T.astype(jnp.bfloat16))
    boutp = b_out.reshape(1, dim)
    cbT = jnp.asarray(_CBT, jnp.bfloat16)

    grid = ntok // _T
    out, idx, aux = pl.pallas_call(
        _body,
        grid=(grid,),
        in_specs=[
            pl.BlockSpec((_T, dim), lambda i: (i, 0)),
            pl.BlockSpec((dim, _DP), lambda i: (0, 0)),
            pl.BlockSpec((1, _DP), lambda i: (0, 0)),
            pl.BlockSpec((_DP, _KP), lambda i: (0, 0)),
            pl.BlockSpec((_DP, dim), lambda i: (0, 0)),
            pl.BlockSpec((1, dim), lambda i: (0, 0)),
            pl.BlockSpec((8, _DP), lambda i: (0, 0)),
            pl.BlockSpec((_DP, 8), lambda i: (0, 0)),
            pl.BlockSpec((_DP, 8), lambda i: (0, 0)),
        ],
        out_specs=[
            pl.BlockSpec((_T, dim), lambda i: (i, 0)),
            pl.BlockSpec((_T, 1), lambda i: (i, 0)),
            pl.BlockSpec((1, 1), lambda i: (0, 0)),
        ],
        out_shape=[
            jax.ShapeDtypeStruct((ntok, dim), jnp.float32),
            jax.ShapeDtypeStruct((ntok, 1), jnp.int32),
            jax.ShapeDtypeStruct((1, 1), jnp.float32),
        ],
        scratch_shapes=[
            pltpu.VMEM((1, _KP), jnp.float32),
            pltpu.SMEM((1,), jnp.float32),
        ],
        compiler_params=pltpu.CompilerParams(
            dimension_semantics=("arbitrary",)),
    )(zf, winT, binp, cbT, woutT, boutp,
      jnp.asarray(_QC), jnp.asarray(_CS), jnp.asarray(_MS))
    return out.reshape(b, n, dim), idx.reshape(b, n), aux[0, 0]
